# baseline (speedup 1.0000x reference)
import functools

import jax
import jax.numpy as jnp
from jax import lax
from jax.experimental import pallas as pl
from jax.experimental.pallas import tpu as pltpu
from jax.experimental.pallas import tpu_sc as plsc

NC = 2
NS = 16
NW = NC * NS
L = 16

R, C = 2048, 2048
ROWS_W = R // NW
ROWS_CH = 8
NCH = ROWS_W // ROWS_CH
CVEC = C // L

NBUCKET = 2048
BVEC = NBUCKET // L

_f32 = jnp.float32
_i32 = jnp.int32


def _worker_id():
    return lax.axis_index("c") * NS + lax.axis_index("s")


@functools.lru_cache(maxsize=None)
def _make_minmax_kernel():
    @functools.partial(
        pl.kernel,
        out_type=(
            jax.ShapeDtypeStruct((NW, L), _f32),
            jax.ShapeDtypeStruct((NW, L), _f32),
        ),
        mesh=plsc.VectorSubcoreMesh(core_axis_name="c", subcore_axis_name="s",
                                    num_cores=NC, num_subcores=NS),
        compiler_params=pltpu.CompilerParams(needs_layout_passes=False),
        scratch_types=[
            pltpu.VMEM((ROWS_CH, C), _f32),
            pltpu.VMEM((ROWS_CH, C), _f32),
            pltpu.VMEM((L,), _f32),
            pltpu.VMEM((L,), _f32),
            pltpu.SemaphoreType.DMA,
            pltpu.SemaphoreType.DMA,
        ],
    )
    def minmax(x_hbm, mn_hbm, mx_hbm, xb0, xb1, mnv, mxv, si0, si1):
        wid = _worker_id()
        row0 = wid * ROWS_W
        xb = (xb0, xb1)
        si = (si0, si1)

        def start_in(ch, b):
            return pltpu.async_copy(
                x_hbm.at[pl.ds(row0 + ch * ROWS_CH, ROWS_CH), :], xb[b],
                si[b])

        start_in(0, 0)
        start_in(1, 1)

        def pair(k, carry):
            for b in range(2):
                ch = 2 * k + b
                pltpu.make_async_copy(
                    x_hbm.at[pl.ds(row0, ROWS_CH), :], xb[b], si[b]).wait()

                def vbody(i, c2):
                    mn, mx = c2
                    for r in range(ROWS_CH):
                        xv = xb[b][r, pl.ds(i * L, L)]
                        mn = jnp.minimum(mn, xv)
                        mx = jnp.maximum(mx, xv)
                    return mn, mx

                carry = lax.fori_loop(0, CVEC, vbody, carry)

                @pl.when(ch + 2 < NCH)
                def _():
                    start_in(ch + 2, b)

            return carry

        init = (jnp.full((L,), jnp.inf, _f32), jnp.full((L,), -jnp.inf, _f32))
        mn, mx = lax.fori_loop(0, NCH // 2, pair, init)
        mnv[...] = mn
        mxv[...] = mx
        pltpu.sync_copy(mnv, mn_hbm.at[wid])
        pltpu.sync_copy(mxv, mx_hbm.at[wid])

    return minmax


@functools.lru_cache(maxsize=None)
def _make_quant_kernel():
    @functools.partial(
        pl.kernel,
        out_type=jax.ShapeDtypeStruct((R, C), _f32),
        mesh=plsc.VectorSubcoreMesh(core_axis_name="c", subcore_axis_name="s",
                                    num_cores=NC, num_subcores=NS),
        compiler_params=pltpu.CompilerParams(needs_layout_passes=False),
        scratch_types=[
            pltpu.VMEM((ROWS_CH, C), _f32),
            pltpu.VMEM((ROWS_CH, C), _f32),
            pltpu.VMEM((ROWS_CH, C), _f32),
            pltpu.VMEM((ROWS_CH, C), _f32),
            pltpu.VMEM((NBUCKET,), _f32),
            pltpu.VMEM((L,), _f32),
            pltpu.VMEM((L,), _f32),
            pltpu.VMEM((NW, L), _f32),
            pltpu.VMEM((NW, L), _f32),
            pltpu.SemaphoreType.DMA,
            pltpu.SemaphoreType.DMA,
            pltpu.SemaphoreType.DMA,
            pltpu.SemaphoreType.DMA,
        ],
    )
    def quant(x_hbm, params_hbm, slv_hbm, mn_hbm, mx_hbm, out_hbm,
              xb0, xb1, ob0, ob1, lutv, pv, slvv, pmn, pmx,
              si0, si1, so0, so1):
        row0 = _worker_id() * ROWS_W
        xb = (xb0, xb1)
        ob = (ob0, ob1)
        si = (si0, si1)
        so = (so0, so1)

        pltpu.sync_copy(params_hbm, pv)
        pltpu.sync_copy(slv_hbm, slvv)
        pvec = pv[...]
        aligned = pvec[0] > 0.5
        mids = [pvec[1 + k] for k in range(8)]

        def bounds():
            pltpu.sync_copy(mn_hbm, pmn)
            pltpu.sync_copy(mx_hbm, pmx)
            mnvec = pmn[0, pl.ds(0, L)]
            mxvec = pmx[0, pl.ds(0, L)]
            for w in range(1, NW):
                mnvec = jnp.minimum(mnvec, pmn[w, pl.ds(0, L)])
                mxvec = jnp.maximum(mxvec, pmx[w, pl.ds(0, L)])
            return jnp.min(mnvec), jnp.max(mxvec)

        def classify(xv, xmin, xmax):
            t = jnp.zeros((L,), _i32)
            for m in mids:
                t = t + jnp.where(xv > m, 1, 0)
            q = plsc.load_gather(slvv, [t])
            return jnp.minimum(jnp.maximum(q, xmin), xmax)

        def stream(compute_chunk, prelude):
            def start_in(ch, b):
                return pltpu.async_copy(
                    x_hbm.at[pl.ds(row0 + ch * ROWS_CH, ROWS_CH), :], xb[b],
                    si[b])

            start_in(0, 0)
            prelude()
            start_in(1, 1)

            def pair(k, _):
                for b in range(2):
                    ch = 2 * k + b
                    pltpu.make_async_copy(
                        x_hbm.at[pl.ds(row0, ROWS_CH), :], xb[b],
                        si[b]).wait()

                    @pl.when(ch >= 2)
                    def _():
                        pltpu.make_async_copy(
                            ob[b], out_hbm.at[pl.ds(row0, ROWS_CH), :],
                            so[b]).wait()

                    compute_chunk(xb[b], ob[b])

                    @pl.when(ch + 2 < NCH)
                    def _():
                        start_in(ch + 2, b)

                    pltpu.async_copy(
                        ob[b],
                        out_hbm.at[pl.ds(row0 + ch * ROWS_CH, ROWS_CH), :],
                        so[b])
                return 0

            lax.fori_loop(0, NCH // 2, pair, 0)
            for b in range(2):
                pltpu.make_async_copy(
                    ob[b], out_hbm.at[pl.ds(row0, ROWS_CH), :], so[b]).wait()

        def prep_bucket():
            xmin, xmax = bounds()
            base = lax.iota(_i32, L)

            @plsc.parallel_loop(0, BVEC)
            def _(i):
                bits = ((i * L + base) << 21) | 0x100000
                rep = plsc.bitcast(bits, _f32)
                lutv[pl.ds(i * L, L)] = classify(rep, xmin, xmax)

        def bucket_chunk(cxb, cob):
            @plsc.parallel_loop(0, CVEC, unroll=4)
            def _(i):
                for r in range(ROWS_CH):
                    xv = cxb[r, pl.ds(i * L, L)]
                    t = jnp.right_shift(plsc.bitcast(xv, _i32), 21) & 0x7FF
                    cob[r, pl.ds(i * L, L)] = plsc.load_gather(lutv, [t])

        gen_state = {}

        def prep_gen():
            gen_state["b"] = bounds()

        def gen_chunk(cxb, cob):
            xmin, xmax = gen_state["b"]

            def vbody(i, _):
                for r in range(ROWS_CH):
                    xv = cxb[r, pl.ds(i * L, L)]
                    cob[r, pl.ds(i * L, L)] = classify(xv, xmin, xmax)
                return 0

            lax.fori_loop(0, CVEC, vbody, 0)

        @pl.when(aligned)
        def _():
            stream(bucket_chunk, prep_bucket)

        @pl.when(jnp.logical_not(aligned))
        def _():
            stream(gen_chunk, prep_gen)

    return quant


def kernel(x, scales):
    s0 = scales[0, 0]
    s1 = scales[1, 0]
    vals = jnp.array([-1.0, 0.0, 1.0], _f32)
    levels = (vals[:, None] * s0 + vals[None, :] * s1).ravel()
    slv = jnp.sort(levels)
    mids = 0.5 * (slv[1:] + slv[:-1])
    mbits = jax.lax.bitcast_convert_type(mids, _i32)
    aligned = jnp.all((mbits & 0x1FFFFF) == 0)
    flag = jnp.where(aligned, 1.0, 0.0).astype(_f32)

    mn, mx = _make_minmax_kernel()(x)
    params = jnp.concatenate([flag[None], mids, jnp.zeros((7,), _f32)])
    slv16 = jnp.concatenate([slv, jnp.full((L - 9,), slv[8], _f32)])

    out = _make_quant_kernel()(x, params, slv16, mn, mx)
    return out


# speedup vs baseline: 1.0031x; 1.0031x over previous
import functools

import jax
import jax.numpy as jnp
from jax import lax
from jax.experimental import pallas as pl
from jax.experimental.pallas import tpu as pltpu
from jax.experimental.pallas import tpu_sc as plsc

NC = 2
NS = 16
NW = NC * NS
L = 16

R, C = 2048, 2048
ROWS_W = R // NW
ROWS_CH = 8
NCH = ROWS_W // ROWS_CH
CVEC = C // L

NBUCKET = 2048
BVEC = NBUCKET // L

_f32 = jnp.float32
_i32 = jnp.int32


def _worker_id():
    return lax.axis_index("c") * NS + lax.axis_index("s")


@functools.lru_cache(maxsize=None)
def _make_minmax_kernel():
    @functools.partial(
        pl.kernel,
        out_type=(
            jax.ShapeDtypeStruct((NW, L), _f32),
            jax.ShapeDtypeStruct((NW, L), _f32),
        ),
        mesh=plsc.VectorSubcoreMesh(core_axis_name="c", subcore_axis_name="s",
                                    num_cores=NC, num_subcores=NS),
        compiler_params=pltpu.CompilerParams(needs_layout_passes=False),
        scratch_types=[
            pltpu.VMEM((ROWS_CH, C), _f32),
            pltpu.VMEM((ROWS_CH, C), _f32),
            pltpu.VMEM((L,), _f32),
            pltpu.VMEM((L,), _f32),
            pltpu.SemaphoreType.DMA,
            pltpu.SemaphoreType.DMA,
        ],
    )
    def minmax(x_hbm, mn_hbm, mx_hbm, xb0, xb1, mnv, mxv, si0, si1):
        wid = _worker_id()
        row0 = wid * ROWS_W
        xb = (xb0, xb1)
        si = (si0, si1)

        def start_in(ch, b):
            return pltpu.async_copy(
                x_hbm.at[pl.ds(row0 + ch * ROWS_CH, ROWS_CH), :], xb[b],
                si[b])

        start_in(0, 0)
        start_in(1, 1)

        def pair(k, carry):
            for b in range(2):
                ch = 2 * k + b
                pltpu.make_async_copy(
                    x_hbm.at[pl.ds(row0, ROWS_CH), :], xb[b], si[b]).wait()

                def vbody(i, c2):
                    mn, mx = c2
                    for r in range(ROWS_CH):
                        xv = xb[b][r, pl.ds(i * L, L)]
                        mn = jnp.minimum(mn, xv)
                        mx = jnp.maximum(mx, xv)
                    return mn, mx

                carry = lax.fori_loop(0, CVEC, vbody, carry)

                @pl.when(ch + 2 < NCH)
                def _():
                    start_in(ch + 2, b)

            return carry

        init = (jnp.full((L,), jnp.inf, _f32), jnp.full((L,), -jnp.inf, _f32))
        mn, mx = lax.fori_loop(0, NCH // 2, pair, init)
        mnv[...] = mn
        mxv[...] = mx
        pltpu.sync_copy(mnv, mn_hbm.at[wid])
        pltpu.sync_copy(mxv, mx_hbm.at[wid])

    return minmax


@functools.lru_cache(maxsize=None)
def _make_quant_kernel():
    @functools.partial(
        pl.kernel,
        out_type=jax.ShapeDtypeStruct((R, C), _f32),
        mesh=plsc.VectorSubcoreMesh(core_axis_name="c", subcore_axis_name="s",
                                    num_cores=NC, num_subcores=NS),
        compiler_params=pltpu.CompilerParams(needs_layout_passes=False),
        scratch_types=[
            pltpu.VMEM((ROWS_CH, C), _f32),
            pltpu.VMEM((ROWS_CH, C), _f32),
            pltpu.VMEM((ROWS_CH, C), _f32),
            pltpu.VMEM((ROWS_CH, C), _f32),
            pltpu.VMEM((NBUCKET,), _f32),
            pltpu.VMEM((L,), _f32),
            pltpu.VMEM((L,), _f32),
            pltpu.VMEM((NW, L), _f32),
            pltpu.VMEM((NW, L), _f32),
            pltpu.SemaphoreType.DMA,
            pltpu.SemaphoreType.DMA,
            pltpu.SemaphoreType.DMA,
            pltpu.SemaphoreType.DMA,
        ],
    )
    def quant(x_hbm, params_hbm, slv_hbm, mn_hbm, mx_hbm, out_hbm,
              xb0, xb1, ob0, ob1, lutv, pv, slvv, pmn, pmx,
              si0, si1, so0, so1):
        row0 = _worker_id() * ROWS_W
        xb = (xb0, xb1)
        ob = (ob0, ob1)
        si = (si0, si1)
        so = (so0, so1)

        pltpu.sync_copy(params_hbm, pv)
        pltpu.sync_copy(slv_hbm, slvv)
        pvec = pv[...]
        aligned = pvec[0] > 0.5
        mids = [pvec[1 + k] for k in range(8)]

        def bounds():
            pltpu.sync_copy(mn_hbm, pmn)
            pltpu.sync_copy(mx_hbm, pmx)
            mnvec = pmn[0, pl.ds(0, L)]
            mxvec = pmx[0, pl.ds(0, L)]
            for w in range(1, NW):
                mnvec = jnp.minimum(mnvec, pmn[w, pl.ds(0, L)])
                mxvec = jnp.maximum(mxvec, pmx[w, pl.ds(0, L)])
            return jnp.min(mnvec), jnp.max(mxvec)

        def classify(xv, xmin, xmax):
            t = jnp.zeros((L,), _i32)
            for m in mids:
                t = t + jnp.where(xv > m, 1, 0)
            q = plsc.load_gather(slvv, [t])
            return jnp.minimum(jnp.maximum(q, xmin), xmax)

        def stream(compute_chunk, prelude):
            def start_in(ch, b):
                return pltpu.async_copy(
                    x_hbm.at[pl.ds(row0 + ch * ROWS_CH, ROWS_CH), :], xb[b],
                    si[b])

            start_in(0, 0)
            prelude()
            start_in(1, 1)

            def pair(k, _):
                for b in range(2):
                    ch = 2 * k + b
                    pltpu.make_async_copy(
                        x_hbm.at[pl.ds(row0, ROWS_CH), :], xb[b],
                        si[b]).wait()

                    @pl.when(ch >= 2)
                    def _():
                        pltpu.make_async_copy(
                            ob[b], out_hbm.at[pl.ds(row0, ROWS_CH), :],
                            so[b]).wait()

                    compute_chunk(xb[b], ob[b])

                    @pl.when(ch + 2 < NCH)
                    def _():
                        start_in(ch + 2, b)

                    pltpu.async_copy(
                        ob[b],
                        out_hbm.at[pl.ds(row0 + ch * ROWS_CH, ROWS_CH), :],
                        so[b])
                return 0

            lax.fori_loop(0, NCH // 2, pair, 0)
            for b in range(2):
                pltpu.make_async_copy(
                    ob[b], out_hbm.at[pl.ds(row0, ROWS_CH), :], so[b]).wait()

        def prep_bucket():
            xmin, xmax = bounds()
            base = lax.iota(_i32, L)

            @plsc.parallel_loop(0, BVEC)
            def _(i):
                bits = ((i * L + base) << 21) | 0x100000
                rep = plsc.bitcast(bits, _f32)
                lutv[pl.ds(i * L, L)] = classify(rep, xmin, xmax)

        def bucket_chunk(cxb, cob):
            @plsc.parallel_loop(0, CVEC, unroll=2)
            def _(i):
                for r in range(ROWS_CH):
                    xv = cxb[r, pl.ds(i * L, L)]
                    t = jnp.right_shift(plsc.bitcast(xv, _i32), 21) & 0x7FF
                    cob[r, pl.ds(i * L, L)] = plsc.load_gather(lutv, [t])

        gen_state = {}

        def prep_gen():
            gen_state["b"] = bounds()

        def gen_chunk(cxb, cob):
            xmin, xmax = gen_state["b"]

            def vbody(i, _):
                for r in range(ROWS_CH):
                    xv = cxb[r, pl.ds(i * L, L)]
                    cob[r, pl.ds(i * L, L)] = classify(xv, xmin, xmax)
                return 0

            lax.fori_loop(0, CVEC, vbody, 0)

        @pl.when(aligned)
        def _():
            stream(bucket_chunk, prep_bucket)

        @pl.when(jnp.logical_not(aligned))
        def _():
            stream(gen_chunk, prep_gen)

    return quant


def kernel(x, scales):
    s0 = scales[0, 0]
    s1 = scales[1, 0]
    vals = jnp.array([-1.0, 0.0, 1.0], _f32)
    levels = (vals[:, None] * s0 + vals[None, :] * s1).ravel()
    slv = jnp.sort(levels)
    mids = 0.5 * (slv[1:] + slv[:-1])
    mbits = jax.lax.bitcast_convert_type(mids, _i32)
    aligned = jnp.all((mbits & 0x1FFFFF) == 0)
    flag = jnp.where(aligned, 1.0, 0.0).astype(_f32)

    mn, mx = _make_minmax_kernel()(x)
    params = jnp.concatenate([flag[None], mids, jnp.zeros((7,), _f32)])
    slv16 = jnp.concatenate([slv, jnp.full((L - 9,), slv[8], _f32)])

    out = _make_quant_kernel()(x, params, slv16, mn, mx)
    return out
